# Initial kernel scaffold; baseline (speedup 1.0000x reference)
import jax
import jax.numpy as jnp
from jax.experimental import pallas as pl


def kernel(x):
    raise NotImplementedError("write your pallas kernel here")


# speedup vs baseline: 6.4785x; 6.4785x over previous
import jax
import jax.numpy as jnp
from jax import lax
from jax.experimental import pallas as pl
from jax.experimental.pallas import tpu as pltpu
from jax.experimental.pallas import tpu_sc as plsc

ROWS = 128
COLS = 32768
KTOP = 256
LANES = 16
NUM_CORES = 2
NUM_SUBCORES = 16
NWORK = NUM_CORES * NUM_SUBCORES
ROWS_PER_W = ROWS // NWORK
NV = COLS // LANES
BINS = 256
HIST_STRIDE = BINS + 1
HIST_WORDS = LANES * HIST_STRIDE


def _topk_mask_body(xi_hbm, out_hbm, xrow, kbuf, hist):
    wid = lax.axis_index("s") * NUM_CORES + lax.axis_index("c")
    lane_off = lax.iota(jnp.int32, LANES) * HIST_STRIDE
    ones = jnp.ones((LANES,), jnp.int32)
    zeros16 = jnp.zeros((LANES,), jnp.int32)

    def do_row(r, rc):
        row = wid * ROWS_PER_W + r
        pltpu.sync_copy(xi_hbm.at[row], xrow)

        def clear_hist(i, c):
            hist[pl.ds(i * LANES, LANES)] = zeros16
            return c

        def hist_pass(shift, prefix_vec):
            def body(i, c):
                if shift == 24:
                    b = xrow[pl.ds(i * LANES, LANES)]
                    s = b ^ ((b >> 31) & jnp.int32(0x7FFFFFFF))
                    kbuf[pl.ds(i * LANES, LANES)] = s
                    mask = None
                    digit = (s >> 24) + jnp.int32(128)
                else:
                    s = kbuf[pl.ds(i * LANES, LANES)]
                    mask = (s >> (shift + 8)) == prefix_vec
                    digit = (s >> shift) & jnp.int32(BINS - 1)
                plsc.addupdate_scatter(hist, [digit + lane_off], ones, mask=mask)
                return c

            return body

        prefix = jnp.zeros((LANES,), jnp.int32)
        k_rem = jnp.int32(KTOP)
        for shift in (24, 16, 8, 0):
            lax.fori_loop(0, HIST_STRIDE, clear_hist, 0)
            lax.fori_loop(0, NV, hist_pass(shift, prefix), 0)
            carry = jnp.int32(0)
            above = jnp.int32(0)
            d_count = jnp.zeros((LANES,), jnp.int32)
            for j in range(BINS // LANES - 1, -1, -1):
                v = hist[pl.ds(j * LANES, LANES)]
                for l in range(1, LANES):
                    v = v + hist[pl.ds(l * HIST_STRIDE + j * LANES, LANES)]
                sfx = lax.rev(plsc.cumsum(lax.rev(v, (0,))), (0,)) + carry
                ge = sfx >= k_rem
                d_count = d_count + plsc.all_reduce_population_count(ge)
                above = above + jnp.sum(jnp.where(ge, 0, v))
                carry = carry + jnp.sum(v)
            dsel = d_count - 1
            if shift == 24:
                prefix = dsel - jnp.int32(128)
            else:
                prefix = (prefix << 8) | dsel
            k_rem = k_rem - above

        tvec = prefix

        def mask_pass(i, c):
            s = kbuf[pl.ds(i * LANES, LANES)]
            b = xrow[pl.ds(i * LANES, LANES)]
            xrow[pl.ds(i * LANES, LANES)] = jnp.where(s >= tvec, b, jnp.int32(0))
            return c

        lax.fori_loop(0, NV, mask_pass, 0)
        pltpu.sync_copy(xrow, out_hbm.at[row])
        return rc

    lax.fori_loop(0, ROWS_PER_W, do_row, 0)


@jax.jit
def kernel(x):
    mesh = plsc.VectorSubcoreMesh(
        core_axis_name="c", subcore_axis_name="s",
        num_cores=NUM_CORES, num_subcores=NUM_SUBCORES,
    )
    run = pl.kernel(
        _topk_mask_body,
        out_type=jax.ShapeDtypeStruct((ROWS, COLS), jnp.int32),
        mesh=mesh,
        compiler_params=pltpu.CompilerParams(needs_layout_passes=False),
        scratch_types=[
            pltpu.VMEM((COLS,), jnp.int32),
            pltpu.VMEM((COLS,), jnp.int32),
            pltpu.VMEM((HIST_WORDS,), jnp.int32),
        ],
    )
    xi = lax.bitcast_convert_type(x, jnp.int32)
    return lax.bitcast_convert_type(run(xi), jnp.float32)


# speedup vs baseline: 21.4894x; 3.3170x over previous
import jax
import jax.numpy as jnp
from jax import lax
from jax.experimental import pallas as pl
from jax.experimental.pallas import tpu as pltpu
from jax.experimental.pallas import tpu_sc as plsc

ROWS = 128
COLS = 32768
KTOP = 256
LANES = 16
NUM_CORES = 2
NUM_SUBCORES = 16
NWORK = NUM_CORES * NUM_SUBCORES
ROWS_PER_W = ROWS // NWORK
NV = COLS // LANES
BINS = 256
NGROUPS = BINS // LANES
HIST_STRIDE = BINS + 1
HIST_WORDS = LANES * HIST_STRIDE
UNROLL = 4


def _tree_add(vs):
    while len(vs) > 1:
        vs = [a + b for a, b in zip(vs[::2], vs[1::2])]
    return vs[0]


def _suffix(v):
    return lax.rev(plsc.cumsum(lax.rev(v, (0,))), (0,))


def _topk_mask_body(xi_hbm, out_hbm, xrow, kbuf, hist):
    wid = lax.axis_index("s") * NUM_CORES + lax.axis_index("c")
    lane_iota = lax.iota(jnp.int32, LANES)
    lane_off = lane_iota * HIST_STRIDE
    ones = jnp.ones((LANES,), jnp.int32)
    zeros16 = jnp.zeros((LANES,), jnp.int32)

    def do_row(r, rc):
        row = wid * ROWS_PER_W + r
        pltpu.sync_copy(xi_hbm.at[row], xrow)

        def select_bin(k_rem):
            gv = zeros16
            for j in range(NGROUPS):
                tot = _tree_add([hist[pl.ds(l * HIST_STRIDE + j * LANES, LANES)]
                                 for l in range(LANES)])
                gv = jnp.where(lane_iota == j, jnp.sum(tot), gv)
            sfxg = _suffix(gv)
            geg = sfxg >= k_rem
            grp = jnp.sum(geg.astype(jnp.int32)) - 1
            above_g = jnp.sum(jnp.where(geg, 0, gv))
            tot = _tree_add([hist[pl.ds(l * HIST_STRIDE + grp * LANES, LANES)]
                             for l in range(LANES)])
            sfx = _suffix(tot) + above_g
            ge = sfx >= k_rem
            dsel = grp * LANES + jnp.sum(ge.astype(jnp.int32)) - 1
            above = above_g + jnp.sum(jnp.where(ge, 0, tot))
            return dsel, above

        prefix = jnp.int32(0)
        k_rem = jnp.int32(KTOP)
        for shift in (24, 16, 8, 0):
            @plsc.parallel_loop(0, HIST_STRIDE, unroll=UNROLL)
            def clear_hist(i):
                hist[pl.ds(i * LANES, LANES)] = zeros16

            @plsc.parallel_loop(0, NV, unroll=UNROLL)
            def hist_pass(i):
                if shift == 24:
                    b = xrow[pl.ds(i * LANES, LANES)]
                    s = b ^ ((b >> 31) & jnp.int32(0x7FFFFFFF))
                    kbuf[pl.ds(i * LANES, LANES)] = s
                    mask = None
                    digit = (s >> 24) + jnp.int32(128)
                else:
                    s = kbuf[pl.ds(i * LANES, LANES)]
                    mask = (s >> (shift + 8)) == prefix
                    digit = (s >> shift) & jnp.int32(BINS - 1)
                plsc.addupdate_scatter(hist, [digit + lane_off], ones, mask=mask)

            dsel, above = select_bin(k_rem)
            if shift == 24:
                prefix = dsel - jnp.int32(128)
            else:
                prefix = (prefix << 8) | dsel
            k_rem = k_rem - above

        tvec = prefix

        @plsc.parallel_loop(0, NV, unroll=UNROLL)
        def mask_pass(i):
            s = kbuf[pl.ds(i * LANES, LANES)]
            b = xrow[pl.ds(i * LANES, LANES)]
            xrow[pl.ds(i * LANES, LANES)] = jnp.where(s >= tvec, b, jnp.int32(0))

        pltpu.sync_copy(xrow, out_hbm.at[row])
        return rc

    lax.fori_loop(0, ROWS_PER_W, do_row, 0)


@jax.jit
def kernel(x):
    mesh = plsc.VectorSubcoreMesh(
        core_axis_name="c", subcore_axis_name="s",
        num_cores=NUM_CORES, num_subcores=NUM_SUBCORES,
    )
    run = pl.kernel(
        _topk_mask_body,
        out_type=jax.ShapeDtypeStruct((ROWS, COLS), jnp.int32),
        mesh=mesh,
        compiler_params=pltpu.CompilerParams(needs_layout_passes=False),
        scratch_types=[
            pltpu.VMEM((COLS,), jnp.int32),
            pltpu.VMEM((COLS,), jnp.int32),
            pltpu.VMEM((HIST_WORDS,), jnp.int32),
        ],
    )
    xi = lax.bitcast_convert_type(x, jnp.int32)
    return lax.bitcast_convert_type(run(xi), jnp.float32)


# speedup vs baseline: 23.9939x; 1.1165x over previous
import jax
import jax.numpy as jnp
from jax import lax
from jax.experimental import pallas as pl
from jax.experimental.pallas import tpu as pltpu
from jax.experimental.pallas import tpu_sc as plsc

ROWS = 128
COLS = 32768
KTOP = 256
LANES = 16
NUM_CORES = 2
NUM_SUBCORES = 16
NWORK = NUM_CORES * NUM_SUBCORES
ROWS_PER_W = ROWS // NWORK
NV = COLS // LANES
BINS = 256
NGROUPS = BINS // LANES
HIST_STRIDE = BINS + 1
HIST_WORDS = LANES * HIST_STRIDE
UNROLL = 4


def _tree_add(vs):
    while len(vs) > 1:
        vs = [a + b for a, b in zip(vs[::2], vs[1::2])]
    return vs[0]


def _suffix(v):
    return lax.rev(plsc.cumsum(lax.rev(v, (0,))), (0,))


def _topk_mask_body(xi_hbm, out_hbm, xrow, kbuf, sbuf, hist):
    wid = lax.axis_index("s") * NUM_CORES + lax.axis_index("c")
    lane_iota = lax.iota(jnp.int32, LANES)
    lane_off = lane_iota * HIST_STRIDE
    ones = jnp.ones((LANES,), jnp.int32)
    zeros16 = jnp.zeros((LANES,), jnp.int32)

    def do_row(r, rc):
        row = wid * ROWS_PER_W + r
        pltpu.sync_copy(xi_hbm.at[row], xrow)

        def select_bin(k_rem):
            gv = zeros16
            for j in range(NGROUPS):
                tot = _tree_add([hist[pl.ds(l * HIST_STRIDE + j * LANES, LANES)]
                                 for l in range(LANES)])
                gv = jnp.where(lane_iota == j, jnp.sum(tot), gv)
            sfxg = _suffix(gv)
            geg = sfxg >= k_rem
            grp = jnp.sum(geg.astype(jnp.int32)) - 1
            above_g = jnp.sum(jnp.where(geg, 0, gv))
            tot = _tree_add([hist[pl.ds(l * HIST_STRIDE + grp * LANES, LANES)]
                             for l in range(LANES)])
            sfx = _suffix(tot) + above_g
            ge = sfx >= k_rem
            dsel = grp * LANES + jnp.sum(ge.astype(jnp.int32)) - 1
            above = above_g + jnp.sum(jnp.where(ge, 0, tot))
            return dsel, above

        prefix = jnp.int32(0)
        k_rem = jnp.int32(KTOP)
        n_sv = jnp.int32(0)
        for shift in (24, 16, 8, 0):
            @plsc.parallel_loop(0, HIST_STRIDE, unroll=UNROLL)
            def clear_hist(i):
                hist[pl.ds(i * LANES, LANES)] = zeros16

            if shift == 24:
                @plsc.parallel_loop(0, NV, unroll=UNROLL)
                def hist_pass(i):
                    b = xrow[pl.ds(i * LANES, LANES)]
                    s = b ^ ((b >> 31) & jnp.int32(0x7FFFFFFF))
                    kbuf[pl.ds(i * LANES, LANES)] = s
                    digit = (s >> 24) + jnp.int32(128)
                    plsc.addupdate_scatter(hist, [digit + lane_off], ones)
            elif shift == 16:
                @plsc.parallel_loop(0, NV, unroll=UNROLL, carry=jnp.int32(0))
                def hist_pass(i, off):
                    s = kbuf[pl.ds(i * LANES, LANES)]
                    mask = (s >> 24) == prefix
                    digit = (s >> 16) & jnp.int32(BINS - 1)
                    plsc.addupdate_scatter(hist, [digit + lane_off], ones, mask=mask)
                    plsc.store_compressed(sbuf.at[pl.ds(off, LANES)], s, mask=mask)
                    return off + jnp.sum(mask.astype(jnp.int32))
                n_sv = hist_pass
            else:
                nv_s = (n_sv + (LANES - 1)) // LANES

                @plsc.parallel_loop(0, nv_s, unroll=2)
                def hist_pass(i):
                    s = sbuf[pl.ds(i * LANES, LANES)]
                    valid = (i * LANES + lane_iota) < n_sv
                    mask = valid & ((s >> (shift + 8)) == prefix)
                    digit = (s >> shift) & jnp.int32(BINS - 1)
                    plsc.addupdate_scatter(hist, [digit + lane_off], ones, mask=mask)

            dsel, above = select_bin(k_rem)
            if shift == 24:
                prefix = dsel - jnp.int32(128)
            else:
                prefix = (prefix << 8) | dsel
            k_rem = k_rem - above

        tvec = prefix

        @plsc.parallel_loop(0, NV, unroll=UNROLL)
        def mask_pass(i):
            s = kbuf[pl.ds(i * LANES, LANES)]
            b = xrow[pl.ds(i * LANES, LANES)]
            xrow[pl.ds(i * LANES, LANES)] = jnp.where(s >= tvec, b, jnp.int32(0))

        pltpu.sync_copy(xrow, out_hbm.at[row])
        return rc

    lax.fori_loop(0, ROWS_PER_W, do_row, 0)


@jax.jit
def kernel(x):
    mesh = plsc.VectorSubcoreMesh(
        core_axis_name="c", subcore_axis_name="s",
        num_cores=NUM_CORES, num_subcores=NUM_SUBCORES,
    )
    run = pl.kernel(
        _topk_mask_body,
        out_type=jax.ShapeDtypeStruct((ROWS, COLS), jnp.int32),
        mesh=mesh,
        compiler_params=pltpu.CompilerParams(needs_layout_passes=False),
        scratch_types=[
            pltpu.VMEM((COLS,), jnp.int32),
            pltpu.VMEM((COLS,), jnp.int32),
            pltpu.VMEM((COLS + LANES,), jnp.int32),
            pltpu.VMEM((HIST_WORDS,), jnp.int32),
        ],
    )
    xi = lax.bitcast_convert_type(x, jnp.int32)
    return lax.bitcast_convert_type(run(xi), jnp.float32)


# speedup vs baseline: 26.4912x; 1.1041x over previous
import jax
import jax.numpy as jnp
from jax import lax
from jax.experimental import pallas as pl
from jax.experimental.pallas import tpu as pltpu
from jax.experimental.pallas import tpu_sc as plsc

ROWS = 128
COLS = 32768
KTOP = 256
LANES = 16
NUM_CORES = 2
NUM_SUBCORES = 16
NWORK = NUM_CORES * NUM_SUBCORES
ROWS_PER_W = ROWS // NWORK
NV = COLS // LANES
BINS = 256
NGROUPS = BINS // LANES
HIST_STRIDE = BINS + 1
HIST_WORDS = LANES * HIST_STRIDE
UNROLL = 4


def _tree_add(vs):
    while len(vs) > 1:
        vs = [a + b for a, b in zip(vs[::2], vs[1::2])]
    return vs[0]


def _suffix(v):
    return lax.rev(plsc.cumsum(lax.rev(v, (0,))), (0,))


def _key(b):
    return b ^ ((b >> 31) & jnp.int32(0x7FFFFFFF))


def _topk_mask_body(xi_hbm, out_hbm, abuf, sbuf, hist, totbuf, in_sem, out_sem):
    wid = lax.axis_index("s") * NUM_CORES + lax.axis_index("c")
    lane_iota = lax.iota(jnp.int32, LANES)
    lane_off = lane_iota * HIST_STRIDE
    ones = jnp.ones((LANES,), jnp.int32)
    zeros16 = jnp.zeros((LANES,), jnp.int32)
    row0 = wid * ROWS_PER_W

    @plsc.parallel_loop(0, HIST_STRIDE, unroll=UNROLL)
    def clear0(i):
        hist[pl.ds(i * LANES, LANES)] = zeros16

    pltpu.async_copy(xi_hbm.at[row0], abuf.at[0], in_sem.at[0])

    def do_row(r, rc):
        p = r & 1
        q = 1 - p
        row = row0 + r
        pltpu.make_async_copy(xi_hbm.at[row], abuf.at[p], in_sem.at[p]).wait()

        def select_bin(k_rem):
            gv = zeros16
            for j in range(NGROUPS):
                tot = _tree_add([hist[pl.ds(l * HIST_STRIDE + j * LANES, LANES)]
                                 for l in range(LANES)])
                for l in range(LANES):
                    hist[pl.ds(l * HIST_STRIDE + j * LANES, LANES)] = zeros16
                totbuf[pl.ds(j * LANES, LANES)] = tot
                gv = jnp.where(lane_iota == j, jnp.sum(tot), gv)
            sfxg = _suffix(gv)
            geg = sfxg >= k_rem
            grp = jnp.sum(geg.astype(jnp.int32)) - 1
            above_g = jnp.sum(jnp.where(geg, 0, gv))
            tot = totbuf[pl.ds(grp * LANES, LANES)]
            sfx = _suffix(tot) + above_g
            ge = sfx >= k_rem
            dsel = grp * LANES + jnp.sum(ge.astype(jnp.int32)) - 1
            above = above_g + jnp.sum(jnp.where(ge, 0, tot))
            return dsel, above

        @plsc.parallel_loop(0, NV, unroll=UNROLL)
        def round1(i):
            s = _key(abuf[p, pl.ds(i * LANES, LANES)])
            digit = (s >> 24) + jnp.int32(128)
            plsc.addupdate_scatter(hist, [digit + lane_off], ones)

        dsel, above = select_bin(jnp.int32(KTOP))
        prefix = dsel - jnp.int32(128)
        k_rem = jnp.int32(KTOP) - above

        @plsc.parallel_loop(0, NV, unroll=UNROLL, carry=jnp.int32(0))
        def round2(i, off):
            s = _key(abuf[p, pl.ds(i * LANES, LANES)])
            mask = (s >> 24) == prefix
            digit = (s >> 16) & jnp.int32(BINS - 1)
            plsc.addupdate_scatter(hist, [digit + lane_off], ones, mask=mask)
            plsc.store_compressed(sbuf.at[pl.ds(off, LANES)], s, mask=mask)
            return off + jnp.sum(mask.astype(jnp.int32))

        n_sv = round2
        dsel, above = select_bin(k_rem)
        prefix = (prefix << 8) | dsel
        k_rem = k_rem - above

        @pl.when(r >= 1)
        def _wait_prev_out():
            pltpu.make_async_copy(abuf.at[q], out_hbm.at[row - 1],
                                  out_sem.at[q]).wait()

        @pl.when(r < ROWS_PER_W - 1)
        def _prefetch_next():
            pltpu.async_copy(xi_hbm.at[row + 1], abuf.at[q], in_sem.at[q])

        nv_s = (n_sv + (LANES - 1)) // LANES
        for shift in (8, 0):
            @plsc.parallel_loop(0, nv_s, unroll=2)
            def round34(i):
                s = sbuf[pl.ds(i * LANES, LANES)]
                valid = (i * LANES + lane_iota) < n_sv
                mask = valid & ((s >> (shift + 8)) == prefix)
                digit = (s >> shift) & jnp.int32(BINS - 1)
                plsc.addupdate_scatter(hist, [digit + lane_off], ones, mask=mask)

            dsel, above = select_bin(k_rem)
            prefix = (prefix << 8) | dsel
            k_rem = k_rem - above

        tvec = prefix

        @plsc.parallel_loop(0, NV, unroll=UNROLL)
        def mask_pass(i):
            b = abuf[p, pl.ds(i * LANES, LANES)]
            keep = _key(b) >= tvec
            abuf[p, pl.ds(i * LANES, LANES)] = jnp.where(keep, b, jnp.int32(0))

        pltpu.async_copy(abuf.at[p], out_hbm.at[row], out_sem.at[p])
        return rc

    lax.fori_loop(0, ROWS_PER_W, do_row, 0)
    last = ROWS_PER_W - 1
    pltpu.make_async_copy(abuf.at[last & 1], out_hbm.at[row0 + last],
                          out_sem.at[last & 1]).wait()


@jax.jit
def kernel(x):
    mesh = plsc.VectorSubcoreMesh(
        core_axis_name="c", subcore_axis_name="s",
        num_cores=NUM_CORES, num_subcores=NUM_SUBCORES,
    )
    run = pl.kernel(
        _topk_mask_body,
        out_type=jax.ShapeDtypeStruct((ROWS, COLS), jnp.int32),
        mesh=mesh,
        compiler_params=pltpu.CompilerParams(needs_layout_passes=False),
        scratch_types=[
            pltpu.VMEM((2, COLS), jnp.int32),
            pltpu.VMEM((COLS + LANES,), jnp.int32),
            pltpu.VMEM((HIST_WORDS,), jnp.int32),
            pltpu.VMEM((BINS,), jnp.int32),
            pltpu.SemaphoreType.DMA((2,)),
            pltpu.SemaphoreType.DMA((2,)),
        ],
    )
    xi = lax.bitcast_convert_type(x, jnp.int32)
    return lax.bitcast_convert_type(run(xi), jnp.float32)


# speedup vs baseline: 27.3008x; 1.0306x over previous
import jax
import jax.numpy as jnp
from jax import lax
from jax.experimental import pallas as pl
from jax.experimental.pallas import tpu as pltpu
from jax.experimental.pallas import tpu_sc as plsc

ROWS = 128
COLS = 32768
KTOP = 256
LANES = 16
NUM_CORES = 2
NUM_SUBCORES = 16
NWORK = NUM_CORES * NUM_SUBCORES
ROWS_PER_W = ROWS // NWORK
NV = COLS // LANES
BINS = 256
NGROUPS = BINS // LANES
HIST_STRIDE = BINS + 1
HIST_WORDS = LANES * HIST_STRIDE
UNROLL = 8


def _tree_add(vs):
    while len(vs) > 1:
        vs = [a + b for a, b in zip(vs[::2], vs[1::2])]
    return vs[0]


def _suffix(v):
    return lax.rev(plsc.cumsum(lax.rev(v, (0,))), (0,))


def _key(b):
    return b ^ ((b >> 31) & jnp.int32(0x7FFFFFFF))


def _topk_mask_body(xi_hbm, out_hbm, abuf, sbuf, hist, totbuf, in_sem, out_sem):
    wid = lax.axis_index("s") * NUM_CORES + lax.axis_index("c")
    lane_iota = lax.iota(jnp.int32, LANES)
    lane_off = lane_iota * HIST_STRIDE
    ones = jnp.ones((LANES,), jnp.int32)
    zeros16 = jnp.zeros((LANES,), jnp.int32)
    row0 = wid * ROWS_PER_W

    @plsc.parallel_loop(0, HIST_STRIDE, unroll=UNROLL)
    def clear0(i):
        hist[pl.ds(i * LANES, LANES)] = zeros16

    pltpu.async_copy(xi_hbm.at[row0], abuf.at[0], in_sem.at[0])

    def do_row(r, rc):
        p = r & 1
        q = 1 - p
        row = row0 + r
        pltpu.make_async_copy(xi_hbm.at[row], abuf.at[p], in_sem.at[p]).wait()

        def select_bin(k_rem):
            gv = zeros16
            for j in range(NGROUPS):
                tot = _tree_add([hist[pl.ds(l * HIST_STRIDE + j * LANES, LANES)]
                                 for l in range(LANES)])
                for l in range(LANES):
                    hist[pl.ds(l * HIST_STRIDE + j * LANES, LANES)] = zeros16
                totbuf[pl.ds(j * LANES, LANES)] = tot
                gv = jnp.where(lane_iota == j, jnp.sum(tot), gv)
            sfxg = _suffix(gv)
            geg = sfxg >= k_rem
            grp = jnp.sum(geg.astype(jnp.int32)) - 1
            above_g = jnp.sum(jnp.where(geg, 0, gv))
            tot = totbuf[pl.ds(grp * LANES, LANES)]
            sfx = _suffix(tot) + above_g
            ge = sfx >= k_rem
            dsel = grp * LANES + jnp.sum(ge.astype(jnp.int32)) - 1
            above = above_g + jnp.sum(jnp.where(ge, 0, tot))
            return dsel, above

        @plsc.parallel_loop(0, NV, unroll=UNROLL)
        def round1(i):
            s = _key(abuf[p, pl.ds(i * LANES, LANES)])
            digit = (s >> 24) + jnp.int32(128)
            plsc.addupdate_scatter(hist, [digit + lane_off], ones)

        dsel, above = select_bin(jnp.int32(KTOP))
        prefix = dsel - jnp.int32(128)
        k_rem = jnp.int32(KTOP) - above

        @plsc.parallel_loop(0, NV, unroll=UNROLL, carry=jnp.int32(0))
        def round2(i, off):
            s = _key(abuf[p, pl.ds(i * LANES, LANES)])
            mask = (s >> 24) == prefix
            digit = (s >> 16) & jnp.int32(BINS - 1)
            plsc.addupdate_scatter(hist, [digit + lane_off], ones, mask=mask)
            plsc.store_compressed(sbuf.at[pl.ds(off, LANES)], s, mask=mask)
            cnt = plsc.all_reduce_population_count(mask)
            return off + jnp.squeeze(lax.slice(cnt, (0,), (1,)))

        n_sv = round2
        dsel, above = select_bin(k_rem)
        prefix = (prefix << 8) | dsel
        k_rem = k_rem - above

        @pl.when(r >= 1)
        def _wait_prev_out():
            pltpu.make_async_copy(abuf.at[q], out_hbm.at[row - 1],
                                  out_sem.at[q]).wait()

        @pl.when(r < ROWS_PER_W - 1)
        def _prefetch_next():
            pltpu.async_copy(xi_hbm.at[row + 1], abuf.at[q], in_sem.at[q])

        nv_s = (n_sv + (LANES - 1)) // LANES
        for shift in (8, 0):
            @plsc.parallel_loop(0, nv_s, unroll=2)
            def round34(i):
                s = sbuf[pl.ds(i * LANES, LANES)]
                valid = (i * LANES + lane_iota) < n_sv
                mask = valid & ((s >> (shift + 8)) == prefix)
                digit = (s >> shift) & jnp.int32(BINS - 1)
                plsc.addupdate_scatter(hist, [digit + lane_off], ones, mask=mask)

            dsel, above = select_bin(k_rem)
            prefix = (prefix << 8) | dsel
            k_rem = k_rem - above

        tvec = prefix

        @plsc.parallel_loop(0, NV, unroll=UNROLL)
        def mask_pass(i):
            b = abuf[p, pl.ds(i * LANES, LANES)]
            keep = _key(b) >= tvec
            abuf[p, pl.ds(i * LANES, LANES)] = jnp.where(keep, b, jnp.int32(0))

        pltpu.async_copy(abuf.at[p], out_hbm.at[row], out_sem.at[p])
        return rc

    lax.fori_loop(0, ROWS_PER_W, do_row, 0)
    last = ROWS_PER_W - 1
    pltpu.make_async_copy(abuf.at[last & 1], out_hbm.at[row0 + last],
                          out_sem.at[last & 1]).wait()


@jax.jit
def kernel(x):
    mesh = plsc.VectorSubcoreMesh(
        core_axis_name="c", subcore_axis_name="s",
        num_cores=NUM_CORES, num_subcores=NUM_SUBCORES,
    )
    run = pl.kernel(
        _topk_mask_body,
        out_type=jax.ShapeDtypeStruct((ROWS, COLS), jnp.int32),
        mesh=mesh,
        compiler_params=pltpu.CompilerParams(needs_layout_passes=False),
        scratch_types=[
            pltpu.VMEM((2, COLS), jnp.int32),
            pltpu.VMEM((COLS + LANES,), jnp.int32),
            pltpu.VMEM((HIST_WORDS,), jnp.int32),
            pltpu.VMEM((BINS,), jnp.int32),
            pltpu.SemaphoreType.DMA((2,)),
            pltpu.SemaphoreType.DMA((2,)),
        ],
    )
    xi = lax.bitcast_convert_type(x, jnp.int32)
    return lax.bitcast_convert_type(run(xi), jnp.float32)
